# baseline (speedup 1.0000x reference)
import functools

import jax
import jax.numpy as jnp
from jax.experimental import pallas as pl
from jax.experimental.pallas import tpu as pltpu


def _se_kernel(x_hbm, w1t_ref, b1_ref, w2t_ref, b2_ref, o_hbm,
               in_buf, out_buf, in_sem, out_sem,
               *, n_cores, tiles, B, D, inv_hw):
    core = pl.program_id(0)
    base = core * (tiles * B)

    def start_in(t):
        slot = t % D
        pltpu.make_async_copy(
            x_hbm.at[pl.ds(base + t * B, B)], in_buf.at[slot],
            in_sem.at[slot]).start()

    def wait_in(t):
        slot = t % D
        pltpu.make_async_copy(
            x_hbm.at[pl.ds(base, B)], in_buf.at[slot],
            in_sem.at[slot]).wait()

    def start_out(t):
        slot = t % D
        pltpu.make_async_copy(
            out_buf.at[slot], o_hbm.at[pl.ds(base + t * B, B)],
            out_sem.at[slot]).start(priority=1)

    def wait_out(t):
        slot = t % D
        pltpu.make_async_copy(
            out_buf.at[slot], o_hbm.at[pl.ds(base, B)],
            out_sem.at[slot]).wait()

    for t in range(min(D, tiles)):
        start_in(t)

    for t in range(tiles):
        wait_in(t)
        slot = t % D
        x = in_buf[slot]
        m = jnp.sum(x, axis=2, dtype=jnp.float32) * inv_hw
        z = jnp.dot(m, w1t_ref[...],
                    preferred_element_type=jnp.float32) + b1_ref[...]
        z = jnp.clip(z, 0.0, 6.0)
        e = jnp.dot(z, w2t_ref[...],
                    preferred_element_type=jnp.float32) + b2_ref[...]
        e = jax.nn.sigmoid(e)
        if t >= D:
            wait_out(t - D)
        out_buf[slot] = x * e[:, :, None].astype(x.dtype)
        start_out(t)
        if t + D < tiles:
            start_in(t + D)

    for t in range(max(0, tiles - D), tiles):
        wait_out(t)


def kernel(x, w1, b1, w2, b2):
    N, C, H, W = x.shape
    HW = H * W
    C_mid = w1.shape[0]
    x_flat = x.reshape(N, C, HW)

    w1t = jnp.asarray(w1, jnp.float32).T
    b1r = jnp.asarray(b1, jnp.float32).reshape(1, C_mid)
    w2t = jnp.asarray(w2, jnp.float32).T
    b2r = jnp.asarray(b2, jnp.float32).reshape(1, C)

    n_cores = 2 if N % 2 == 0 else 1
    per_core = N // n_cores
    slab = C * HW * x.dtype.itemsize
    B = 1
    for cand in range(per_core, 0, -1):
        if per_core % cand == 0 and cand * slab <= (2 << 20):
            B = cand
            break
    tiles = per_core // B
    D = min(4, tiles)

    out = pl.pallas_call(
        functools.partial(_se_kernel, n_cores=n_cores, tiles=tiles,
                          B=B, D=D, inv_hw=1.0 / HW),
        out_shape=jax.ShapeDtypeStruct((N, C, HW), x.dtype),
        grid_spec=pltpu.PrefetchScalarGridSpec(
            num_scalar_prefetch=0,
            grid=(n_cores,),
            in_specs=[
                pl.BlockSpec(memory_space=pl.ANY),
                pl.BlockSpec((C, C_mid), lambda c: (0, 0)),
                pl.BlockSpec((1, C_mid), lambda c: (0, 0)),
                pl.BlockSpec((C_mid, C), lambda c: (0, 0)),
                pl.BlockSpec((1, C), lambda c: (0, 0)),
            ],
            out_specs=pl.BlockSpec(memory_space=pl.ANY),
            scratch_shapes=[
                pltpu.VMEM((D, B, C, HW), x.dtype),
                pltpu.VMEM((D, B, C, HW), x.dtype),
                pltpu.SemaphoreType.DMA((D,)),
                pltpu.SemaphoreType.DMA((D,)),
            ],
        ),
        compiler_params=pltpu.CompilerParams(
            dimension_semantics=("parallel",),
            vmem_limit_bytes=int(min(56 << 20, 2 * D * B * slab + (4 << 20)))),
    )(x_flat, w1t, b1r, w2t, b2r)
    return out.reshape(N, C, H, W)


# speedup vs baseline: 1.0866x; 1.0866x over previous
import functools

import jax
import jax.numpy as jnp
from jax.experimental import pallas as pl
from jax.experimental.pallas import tpu as pltpu

_N_PRI = 2


def _se_kernel(x_hbm, w1t_ref, b1_ref, w2t_ref, b2_ref, o_hbm,
               in_buf, out_buf, in_sem, out_sem,
               *, tiles, B, D, inv_hw):
    def start_in(t):
        slot = t % D
        pltpu.make_async_copy(
            x_hbm.at[pl.ds(t * B, B)], in_buf.at[slot],
            in_sem.at[slot]).start(priority=t % _N_PRI)

    def wait_in(t):
        slot = t % D
        pltpu.make_async_copy(
            x_hbm.at[pl.ds(0, B)], in_buf.at[slot],
            in_sem.at[slot]).wait()

    def start_out(t):
        slot = t % D
        pltpu.make_async_copy(
            out_buf.at[slot], o_hbm.at[pl.ds(t * B, B)],
            out_sem.at[slot]).start(priority=t % _N_PRI)

    def wait_out(t):
        slot = t % D
        pltpu.make_async_copy(
            out_buf.at[slot], o_hbm.at[pl.ds(0, B)],
            out_sem.at[slot]).wait()

    for t in range(min(D, tiles)):
        start_in(t)

    for t in range(tiles):
        wait_in(t)
        slot = t % D
        x = in_buf[slot]
        m = jnp.sum(x, axis=2, dtype=jnp.float32) * inv_hw
        z = jnp.dot(m, w1t_ref[...],
                    preferred_element_type=jnp.float32) + b1_ref[...]
        z = jnp.clip(z, 0.0, 6.0)
        e = jnp.dot(z, w2t_ref[...],
                    preferred_element_type=jnp.float32) + b2_ref[...]
        e = jax.nn.sigmoid(e)
        if t >= D:
            wait_out(t - D)
        out_buf[slot] = x * e[:, :, None].astype(x.dtype)
        start_out(t)
        if t + D < tiles:
            start_in(t + D)

    for t in range(max(0, tiles - D), tiles):
        wait_out(t)


def kernel(x, w1, b1, w2, b2):
    N, C, H, W = x.shape
    HW = H * W
    C_mid = w1.shape[0]
    x_flat = x.reshape(N, C, HW)

    w1t = jnp.asarray(w1, jnp.float32).T
    b1r = jnp.asarray(b1, jnp.float32).reshape(1, C_mid)
    w2t = jnp.asarray(w2, jnp.float32).T
    b2r = jnp.asarray(b2, jnp.float32).reshape(1, C)

    slab = C * HW * x.dtype.itemsize
    B = 1
    for cand in range(N, 0, -1):
        if N % cand == 0 and cand * slab <= (2 << 20):
            B = cand
            break
    tiles = N // B
    D = min(6, tiles)

    out = pl.pallas_call(
        functools.partial(_se_kernel, tiles=tiles, B=B, D=D, inv_hw=1.0 / HW),
        out_shape=jax.ShapeDtypeStruct((N, C, HW), x.dtype),
        grid_spec=pltpu.PrefetchScalarGridSpec(
            num_scalar_prefetch=0,
            grid=(1,),
            in_specs=[
                pl.BlockSpec(memory_space=pl.ANY),
                pl.BlockSpec((C, C_mid), lambda c: (0, 0)),
                pl.BlockSpec((1, C_mid), lambda c: (0, 0)),
                pl.BlockSpec((C_mid, C), lambda c: (0, 0)),
                pl.BlockSpec((1, C), lambda c: (0, 0)),
            ],
            out_specs=pl.BlockSpec(memory_space=pl.ANY),
            scratch_shapes=[
                pltpu.VMEM((D, B, C, HW), x.dtype),
                pltpu.VMEM((D, B, C, HW), x.dtype),
                pltpu.SemaphoreType.DMA((D,)),
                pltpu.SemaphoreType.DMA((D,)),
            ],
        ),
        compiler_params=pltpu.CompilerParams(
            dimension_semantics=("arbitrary",),
            vmem_limit_bytes=int(min(56 << 20, 2 * D * B * slab + (4 << 20)))),
    )(x_flat, w1t, b1r, w2t, b2r)
    return out.reshape(N, C, H, W)
